# baseline (speedup 1.0000x reference)
import functools

import jax
import jax.numpy as jnp
from jax import lax
from jax.experimental import pallas as pl
from jax.experimental.pallas import tpu as pltpu
from jax.experimental.pallas import tpu_sc as plsc

_D = 64
_SEQ = 200
_BATCH = 4096
_NC = 2
_NS = 16
_NW = _NC * _NS
_CH = 128
_NBB = _BATCH // _CH
_GROUPS = _SEQ * _NBB
_GPW = _GROUPS // _NW


def _make_gather():
    mesh = plsc.VectorSubcoreMesh(core_axis_name="c", subcore_axis_name="s")

    @functools.partial(
        pl.kernel,
        mesh=mesh,
        out_type=jax.ShapeDtypeStruct((_SEQ, _D, _BATCH), jnp.float32),
        scratch_types=[
            pltpu.VMEM((_CH,), jnp.int32),
            pltpu.VMEM((_CH,), jnp.int32),
            pltpu.VMEM((_CH, 128), jnp.float32),
            pltpu.VMEM((_D, _CH), jnp.float32),
            pltpu.SemaphoreType.DMA,
        ],
        compiler_params=pltpu.CompilerParams(
            use_tc_tiling_on_sc=True, needs_layout_passes=False
        ),
    )
    def emb(xT_hbm, tbl2_hbm, out_hbm, idx_v, u_v, g_v, t_v, sem):
        wid = lax.axis_index("s") * _NC + lax.axis_index("c")
        iota = lax.iota(jnp.int32, 16)

        def body(gi, carry):
            grp = wid * _GPW + gi
            s = grp // _NBB
            bb = grp % _NBB
            pltpu.sync_copy(xT_hbm.at[s, pl.ds(bb * _CH, _CH)], idx_v)
            for r in range(8):
                u_v[pl.ds(r * 16, 16)] = lax.shift_right_logical(
                    idx_v[pl.ds(r * 16, 16)], 1
                )
            pltpu.async_copy(tbl2_hbm.at[u_v], g_v, sem).wait()
            cols = []
            for bv in range(8):
                v = idx_v[pl.ds(bv * 16, 16)]
                cols.append(lax.bitwise_and(v, 1) * _D)
            rows = [bv * 16 + iota for bv in range(8)]

            def tr_body(do, cst):
                rws, cls = cst
                for bv in range(8):
                    val = plsc.load_gather(g_v, [rws[bv], cls[bv] + do])
                    t_v.at[do][pl.ds(bv * 16, 16)] = val
                return cst

            lax.fori_loop(0, _D, tr_body, (tuple(rows), tuple(cols)))
            pltpu.sync_copy(t_v, out_hbm.at[s, :, pl.ds(bb * _CH, _CH)])
            return carry

        lax.fori_loop(0, _GPW, body, 0)

    return emb


_emb_gather = _make_gather()


def kernel(x, table):
    xT = x.T
    tbl2 = table.reshape(500000, 128)
    outT = _emb_gather(xT, tbl2)
    return jnp.transpose(outT, (2, 0, 1))


# speedup vs baseline: 1.1727x; 1.1727x over previous
import functools

import jax
import jax.numpy as jnp
from jax import lax
from jax.experimental import pallas as pl
from jax.experimental.pallas import tpu as pltpu
from jax.experimental.pallas import tpu_sc as plsc

_D = 64
_SEQ = 200
_BATCH = 4096
_NC = 2
_NS = 16
_NW = _NC * _NS
_CH = 128
_NBB = _BATCH // _CH
_GROUPS = _SEQ * _NBB
_GPW = _GROUPS // _NW


def _make_gather():
    mesh = plsc.VectorSubcoreMesh(core_axis_name="c", subcore_axis_name="s")

    @functools.partial(
        pl.kernel,
        mesh=mesh,
        out_type=jax.ShapeDtypeStruct((_SEQ, _D, _BATCH), jnp.float32),
        scratch_types=[
            pltpu.VMEM((2, _CH), jnp.int32),
            pltpu.VMEM((2, _CH), jnp.int32),
            pltpu.VMEM((2, _CH, 128), jnp.float32),
            pltpu.VMEM((2, _D, _CH), jnp.float32),
            pltpu.SemaphoreType.DMA,
            pltpu.SemaphoreType.DMA,
        ],
        compiler_params=pltpu.CompilerParams(
            use_tc_tiling_on_sc=True, needs_layout_passes=False
        ),
    )
    def emb(xT_hbm, tbl2_hbm, out_hbm, idx_v, u_v, g_v, t_v, gsem, osem):
        wid = lax.axis_index("s") * _NC + lax.axis_index("c")
        base = wid * _GPW
        iota = lax.iota(jnp.int32, 16)

        def load_idx(si, grp):
            s = grp // _NBB
            bb = grp % _NBB
            pltpu.sync_copy(xT_hbm.at[s, pl.ds(bb * _CH, _CH)], idx_v.at[si])
            for r in range(8):
                u_v.at[si][pl.ds(r * 16, 16)] = lax.shift_right_logical(
                    idx_v.at[si][pl.ds(r * 16, 16)], 1
                )

        def fire_gather(si):
            pltpu.async_copy(tbl2_hbm.at[u_v.at[si]], g_v.at[si], gsem)

        def drain_gather(si):
            pltpu.make_async_copy(tbl2_hbm.at[u_v.at[si]],
                                  g_v.at[si], gsem).wait()

        def transpose(si):
            rows = [bv * 16 + iota for bv in range(8)]
            cols = [lax.bitwise_and(idx_v.at[si][pl.ds(bv * 16, 16)], 1) * _D
                    for bv in range(8)]

            def dv_body(dv, cst):
                rws, cls = cst
                for di in range(8):
                    do = dv * 8 + di
                    for bv in range(8):
                        val = plsc.load_gather(
                            g_v.at[si], [rws[bv], cls[bv] + do])
                        t_v.at[si].at[do][pl.ds(bv * 16, 16)] = val
                return cst

            lax.fori_loop(0, 8, dv_body, (tuple(rows), tuple(cols)))

        def fire_out(si, grp):
            s = grp // _NBB
            bb = grp % _NBB
            pltpu.async_copy(t_v.at[si],
                             out_hbm.at[s, :, pl.ds(bb * _CH, _CH)], osem)

        def drain_out(si):
            pltpu.make_async_copy(t_v.at[si],
                                  out_hbm.at[0, :, pl.ds(0, _CH)], osem).wait()

        load_idx(0, base)
        fire_gather(0)

        def body(p, carry):
            for st in range(2):
                grp = base + 2 * p + st
                drain_gather(st)
                pl.when(p > 0)(lambda: drain_out(st))
                nxt = base + jnp.minimum(2 * p + st + 1, _GPW - 1)
                load_idx(1 - st, nxt)
                fire_gather(1 - st)
                transpose(st)
                fire_out(st, grp)
            return carry

        lax.fori_loop(0, _GPW // 2, body, 0)
        drain_gather(0)
        drain_out(0)
        drain_out(1)

    return emb


_emb_gather = _make_gather()


def kernel(x, table):
    xT = x.T
    tbl2 = table.reshape(500000, 128)
    outT = _emb_gather(xT, tbl2)
    return jnp.transpose(outT, (2, 0, 1))


# speedup vs baseline: 1.5351x; 1.3090x over previous
import functools

import jax
import jax.numpy as jnp
from jax import lax
from jax.experimental import pallas as pl
from jax.experimental.pallas import tpu as pltpu
from jax.experimental.pallas import tpu_sc as plsc

_D = 64
_SEQ = 200
_BATCH = 4096
_NC = 2
_NS = 16
_NW = _NC * _NS
_CH = 128
_NBB = _BATCH // _CH
_GROUPS = _SEQ * _NBB
_GPW = _GROUPS // _NW


def _make_gather():
    mesh = plsc.VectorSubcoreMesh(core_axis_name="c", subcore_axis_name="s")

    @functools.partial(
        pl.kernel,
        mesh=mesh,
        out_type=jax.ShapeDtypeStruct((_SEQ, _D, _BATCH), jnp.float32),
        scratch_types=[
            pltpu.VMEM((2, _CH), jnp.int32),
            pltpu.VMEM((2, _CH), jnp.int32),
            pltpu.VMEM((2, _CH, 128), jnp.float32),
            pltpu.VMEM((2, _D, _CH), jnp.float32),
            pltpu.SemaphoreType.DMA,
            pltpu.SemaphoreType.DMA,
        ],
        compiler_params=pltpu.CompilerParams(
            use_tc_tiling_on_sc=True, needs_layout_passes=False
        ),
    )
    def emb(xT_hbm, tbl2_hbm, out_hbm, idx_v, u_v, g_v, t_v, gsem, osem):
        wid = lax.axis_index("s") * _NC + lax.axis_index("c")
        base = wid * _GPW
        iota = lax.iota(jnp.int32, 16)

        def load_idx(si, grp):
            s = grp // _NBB
            bb = grp % _NBB
            pltpu.sync_copy(xT_hbm.at[s, pl.ds(bb * _CH, _CH)], idx_v.at[si])
            for r in range(8):
                u_v.at[si][pl.ds(r * 16, 16)] = lax.shift_right_logical(
                    idx_v.at[si][pl.ds(r * 16, 16)], 1
                )

        def fire_gather(si):
            pltpu.async_copy(tbl2_hbm.at[u_v.at[si]], g_v.at[si], gsem)

        def drain_gather(si):
            pltpu.make_async_copy(tbl2_hbm.at[u_v.at[si]],
                                  g_v.at[si], gsem).wait()

        zero16 = iota * 0

        def transpose(si):
            rows = [bv * 16 + iota for bv in range(8)]
            half = [lax.bitwise_and(idx_v.at[si][pl.ds(bv * 16, 16)], 1) * _D
                    for bv in range(8)]
            for bv in range(8):
                for dg in range(_D // 8):
                    vals = [plsc.load_gather(
                        g_v.at[si], [rows[bv], half[bv] + (dg * 8 + k)])
                        for k in range(8)]
                    for k in range(8):
                        t_v.at[si].at[dg * 8 + k][pl.ds(bv * 16, 16)] = vals[k]

        def fire_out(si, grp):
            s = grp // _NBB
            bb = grp % _NBB
            pltpu.async_copy(t_v.at[si],
                             out_hbm.at[s, :, pl.ds(bb * _CH, _CH)], osem)

        def drain_out(si):
            pltpu.make_async_copy(t_v.at[si],
                                  out_hbm.at[0, :, pl.ds(0, _CH)], osem).wait()

        load_idx(0, base)
        fire_gather(0)

        def body(p, carry):
            for st in range(2):
                grp = base + 2 * p + st
                drain_gather(st)
                pl.when(p > 0)(lambda: drain_out(st))
                nxt = base + jnp.minimum(2 * p + st + 1, _GPW - 1)
                load_idx(1 - st, nxt)
                fire_gather(1 - st)
                transpose(st)
                fire_out(st, grp)
            return carry

        lax.fori_loop(0, _GPW // 2, body, 0)
        drain_gather(0)
        drain_out(0)
        drain_out(1)

    return emb


_emb_gather = _make_gather()


def kernel(x, table):
    xT = x.T
    tbl2 = table.reshape(500000, 128)
    outT = _emb_gather(xT, tbl2)
    return jnp.transpose(outT, (2, 0, 1))


# speedup vs baseline: 1.6687x; 1.0870x over previous
import functools

import jax
import jax.numpy as jnp
from jax import lax
from jax.experimental import pallas as pl
from jax.experimental.pallas import tpu as pltpu
from jax.experimental.pallas import tpu_sc as plsc

_D = 64
_SEQ = 200
_BATCH = 4096
_NC = 2
_NS = 16
_NW = _NC * _NS
_CH = 128


def _make_gather():
    mesh = plsc.VectorSubcoreMesh(core_axis_name="c", subcore_axis_name="s")

    @functools.partial(
        pl.kernel,
        mesh=mesh,
        out_type=jax.ShapeDtypeStruct((_SEQ, _D, _BATCH), jnp.float32),
        scratch_types=[
            pltpu.VMEM((_SEQ, _CH), jnp.int32),
            pltpu.VMEM((2, _CH), jnp.int32),
            pltpu.VMEM((2, _CH, 128), jnp.float32),
            pltpu.VMEM((2, _D, _CH), jnp.float32),
            pltpu.SemaphoreType.DMA,
            pltpu.SemaphoreType.DMA,
        ],
        compiler_params=pltpu.CompilerParams(
            use_tc_tiling_on_sc=True, needs_layout_passes=False
        ),
    )
    def emb(xT_hbm, tbl2_hbm, out_hbm, idx_v, u_v, g_v, t_v, gsem, osem):
        wid = lax.axis_index("s") * _NC + lax.axis_index("c")
        iota = lax.iota(jnp.int32, 16)
        pltpu.sync_copy(xT_hbm.at[:, pl.ds(wid * _CH, _CH)], idx_v)

        def fire_gather(st, gi):
            for r in range(8):
                u_v.at[st][pl.ds(r * 16, 16)] = lax.shift_right_logical(
                    idx_v.at[gi][pl.ds(r * 16, 16)], 1
                )
            pltpu.async_copy(tbl2_hbm.at[u_v.at[st]], g_v.at[st], gsem)

        def drain_gather(st):
            pltpu.make_async_copy(tbl2_hbm.at[u_v.at[st]],
                                  g_v.at[st], gsem).wait()

        def transpose(st, gi):
            rows = [bv * 16 + iota for bv in range(8)]
            half = [lax.bitwise_and(idx_v.at[gi][pl.ds(bv * 16, 16)], 1) * _D
                    for bv in range(8)]
            for bv in range(8):
                for dg in range(_D // 8):
                    vals = [plsc.load_gather(
                        g_v.at[st], [rows[bv], half[bv] + (dg * 8 + k)])
                        for k in range(8)]
                    for k in range(8):
                        t_v.at[st].at[dg * 8 + k][pl.ds(bv * 16, 16)] = vals[k]

        def fire_out(st, gi):
            pltpu.async_copy(t_v.at[st],
                             out_hbm.at[gi, :, pl.ds(wid * _CH, _CH)], osem)

        def drain_out(st):
            pltpu.make_async_copy(t_v.at[st],
                                  out_hbm.at[0, :, pl.ds(0, _CH)], osem).wait()

        fire_gather(0, 0)
        fire_gather(1, 1)

        def body(p, carry):
            for st in range(2):
                gi = 2 * p + st
                drain_gather(st)
                pl.when(p > 0)(lambda: drain_out(st))
                transpose(st, gi)
                fire_out(st, gi)
                fire_gather(st, jnp.minimum(gi + 2, _SEQ - 1))
            return carry

        lax.fori_loop(0, _SEQ // 2, body, 0)
        drain_gather(0)
        drain_gather(1)
        drain_out(0)
        drain_out(1)

    return emb


_emb_gather = _make_gather()


def kernel(x, table):
    xT = x.T
    tbl2 = table.reshape(500000, 128)
    outT = _emb_gather(xT, tbl2)
    return jnp.transpose(outT, (2, 0, 1))
